# baseline (speedup 1.0000x reference)
import functools

import jax
import jax.numpy as jnp
from jax import lax
from jax.experimental import pallas as pl
from jax.experimental.pallas import tpu as pltpu
from jax.experimental.pallas import tpu_sc as plsc

_N = 500000
_C = 81
_PC = 80
_NW = 32
_CR = 448
_NCH = 35
_RPW = _CR * _NCH
_MAXSTART = _N - _CR


def _sc_body(probas_hbm, labels_hbm, match_hbm, out_hbm,
             buf, labv, matv, accn, acctp, acccs, stage):
    w = lax.axis_index("s") * 2 + lax.axis_index("c")
    lane = lax.iota(jnp.int32, 16)
    zlane = jnp.zeros((16,), jnp.int32)
    lane128 = lane * 128
    zeros16 = jnp.zeros((16,), jnp.float32)
    ones16 = jnp.ones((16,), jnp.float32)

    for b in range(16):
        accn[pl.ds(b * 16, 16)] = zeros16
        acctp[pl.ds(b * 16, 16)] = zeros16
        acccs[pl.ds(b * 16, 16)] = zeros16

    wbase = w * _RPW

    def chunk_body(c, carry):
        nominal = wbase + c * _CR
        start = jnp.minimum(nominal, _MAXSTART)
        d = nominal - start
        pltpu.sync_copy(probas_hbm.at[pl.ds(start, _CR), :], buf)
        pltpu.sync_copy(labels_hbm.at[pl.ds(start, _CR)], labv)
        pltpu.sync_copy(match_hbm.at[pl.ds(start, _CR)], matv)

        def group_body(g, gcarry):
            local = g * 16 + lane
            ok0 = local >= d
            idx0 = g * (16 * 128) + lane128
            m = plsc.load_gather(buf, [zlane, idx0], mask=ok0)
            idx = idx0
            for _ in range(1, _PC):
                idx = idx + 1
                v = plsc.load_gather(buf, [zlane, idx], mask=ok0)
                m = jnp.maximum(m, v)
            lv = labv[pl.ds(g * 16, 16)]
            mv = matv[pl.ds(g * 16, 16)]
            xl = plsc.load_gather(buf, [zlane, idx0 + lv], mask=ok0)
            mc = jnp.where((xl == m) & (mv != 0), 1.0, 0.0)
            t = m * 10.0
            ti = t.astype(jnp.int32)
            binv = jnp.where(ti.astype(jnp.float32) == t, ti - 1, ti)
            ok = ok0 & (binv >= 0)
            sidx = (binv << 4) + lane
            plsc.addupdate_scatter(accn, [sidx], ones16, mask=ok)
            plsc.addupdate_scatter(acctp, [sidx], mc, mask=ok)
            plsc.addupdate_scatter(acccs, [sidx], m, mask=ok)
            return gcarry

        lax.fori_loop(0, _CR // 16, group_body, 0)
        return carry

    lax.fori_loop(0, _NCH, chunk_body, 0)

    bidx = lane << 4
    vn = zeros16
    vtp = zeros16
    vcs = zeros16
    for l in range(16):
        vn = vn + plsc.load_gather(accn, [bidx + l])
        vtp = vtp + plsc.load_gather(acctp, [bidx + l])
        vcs = vcs + plsc.load_gather(acccs, [bidx + l])
    stage[pl.ds(0, 16)] = vn
    stage[pl.ds(16, 16)] = vtp
    stage[pl.ds(32, 16)] = vcs
    pltpu.sync_copy(stage, out_hbm.at[w])


def _combine(p_ref, out_ref):
    x = p_ref[...]
    n = jnp.sum(x[:, 0:16], axis=0, keepdims=True)
    tp = jnp.sum(x[:, 16:32], axis=0, keepdims=True)
    cs = jnp.sum(x[:, 32:48], axis=0, keepdims=True)
    total = jnp.sum(n)
    nsafe = jnp.maximum(n, 1.0)
    term = jnp.where(n > 0.0, (n / total) * ((cs - tp) / nsafe) ** 2, 0.0)
    out_ref[...] = jnp.full((1, 1), jnp.sqrt(jnp.sum(term)), jnp.float32)


def kernel(probas, labels, matchings):
    labels32 = labels.astype(jnp.int32)
    match32 = matchings.astype(jnp.int32)

    mesh = plsc.VectorSubcoreMesh(core_axis_name="c", subcore_axis_name="s",
                                  num_cores=2, num_subcores=16)
    partials = pl.kernel(
        _sc_body,
        out_type=jax.ShapeDtypeStruct((_NW, 48), jnp.float32),
        mesh=mesh,
        compiler_params=pltpu.CompilerParams(needs_layout_passes=False),
        scratch_types=[
            pltpu.VMEM((_CR, _C), jnp.float32),
            pltpu.VMEM((_CR,), jnp.int32),
            pltpu.VMEM((_CR,), jnp.int32),
            pltpu.VMEM((256,), jnp.float32),
            pltpu.VMEM((256,), jnp.float32),
            pltpu.VMEM((256,), jnp.float32),
            pltpu.VMEM((48,), jnp.float32),
        ],
    )(probas, labels32, match32)

    out = pl.pallas_call(
        _combine,
        in_specs=[pl.BlockSpec((_NW, 48), lambda: (0, 0))],
        out_specs=pl.BlockSpec((1, 1), lambda: (0, 0)),
        out_shape=jax.ShapeDtypeStruct((1, 1), jnp.float32),
    )(partials)
    return out[0, 0]


# speedup vs baseline: 2.7027x; 2.7027x over previous
import functools

import jax
import jax.numpy as jnp
from jax import lax
from jax.experimental import pallas as pl
from jax.experimental.pallas import tpu as pltpu
from jax.experimental.pallas import tpu_sc as plsc

_N = 500000
_C = 81
_PC = 80
_NW = 32
_CR = 448
_NCH = 36
_RPW = _CR * _NCH
_MAXSTART = _N - _CR
_NG = _CR // 16


def _sc_body(probas_hbm, labels_hbm, match_hbm, out_hbm,
             buf0, buf1, lab0, lab1, mat0, mat1,
             accn, acctp, acccs, stage, sem0, sem1):
    w = lax.axis_index("s") * 2 + lax.axis_index("c")
    lane = lax.iota(jnp.int32, 16)
    lane128 = lane * 128
    lane5 = lane * 5
    zeros16 = jnp.zeros((16,), jnp.float32)
    ones16 = jnp.ones((16,), jnp.float32)
    zlane = jnp.zeros((16,), jnp.int32)

    for b in range(16):
        accn[pl.ds(b * 16, 16)] = zeros16
        acctp[pl.ds(b * 16, 16)] = zeros16
        acccs[pl.ds(b * 16, 16)] = zeros16

    wbase = w * _RPW
    bufs = (buf0, buf1)
    labs = (lab0, lab1)
    mats = (mat0, mat1)
    sems = (sem0, sem1)

    def _start(c, p):
        nominal = wbase + c * _CR
        start = jnp.minimum(nominal, _MAXSTART)
        pltpu.async_copy(probas_hbm.at[pl.ds(start, _CR), :], bufs[p], sems[p])
        pltpu.async_copy(labels_hbm.at[pl.ds(start, _CR)], labs[p], sems[p])
        pltpu.async_copy(match_hbm.at[pl.ds(start, _CR)], mats[p], sems[p])

    def _wait(c, p):
        nominal = wbase + c * _CR
        start = jnp.minimum(nominal, _MAXSTART)
        pltpu.make_async_copy(probas_hbm.at[pl.ds(start, _CR), :], bufs[p],
                              sems[p]).wait()
        pltpu.make_async_copy(labels_hbm.at[pl.ds(start, _CR)], labs[p],
                              sems[p]).wait()
        pltpu.make_async_copy(match_hbm.at[pl.ds(start, _CR)], mats[p],
                              sems[p]).wait()

    def _compute(c, p):
        buf, labv, matv = bufs[p], labs[p], mats[p]
        nominal = wbase + c * _CR
        start = jnp.minimum(nominal, _MAXSTART)
        d = nominal - start

        def group_body(g, gcarry):
            local = g * 16 + lane
            ok0 = local >= d
            rowb = g * (16 * 128) + lane128
            colv = lane5
            m = plsc.load_gather(buf, [zlane, rowb + colv], mask=ok0)
            for _ in range(1, _PC):
                colv = colv + 1
                colv = jnp.where(colv == _PC, 0, colv)
                v = plsc.load_gather(buf, [zlane, rowb + colv], mask=ok0)
                m = jnp.maximum(m, v)
            lv = labv[pl.ds(g * 16, 16)]
            mv = matv[pl.ds(g * 16, 16)]
            xl = plsc.load_gather(buf, [zlane, rowb + lv], mask=ok0)
            mc = jnp.where((xl == m) & (mv != 0), 1.0, 0.0)
            t = m * 10.0
            ti = t.astype(jnp.int32)
            binv = jnp.where(ti.astype(jnp.float32) == t, ti - 1, ti)
            ok = ok0 & (binv >= 0)
            sidx = (binv << 4) + lane
            plsc.addupdate_scatter(accn, [sidx], ones16, mask=ok)
            plsc.addupdate_scatter(acctp, [sidx], mc, mask=ok)
            plsc.addupdate_scatter(acccs, [sidx], m, mask=ok)
            return gcarry

        lax.fori_loop(0, _NG, group_body, 0)

    _start(0, 0)

    def pair_body(i, carry):
        c0 = i * 2
        _start(c0 + 1, 1)
        _wait(c0, 0)
        _compute(c0, 0)

        @pl.when(c0 + 2 < _NCH)
        def _():
            _start(c0 + 2, 0)

        _wait(c0 + 1, 1)
        _compute(c0 + 1, 1)
        return carry

    lax.fori_loop(0, _NCH // 2, pair_body, 0)

    bidx = lane << 4
    vn = zeros16
    vtp = zeros16
    vcs = zeros16
    for l in range(16):
        vn = vn + plsc.load_gather(accn, [bidx + l])
        vtp = vtp + plsc.load_gather(acctp, [bidx + l])
        vcs = vcs + plsc.load_gather(acccs, [bidx + l])
    stage[pl.ds(0, 16)] = vn
    stage[pl.ds(16, 16)] = vtp
    stage[pl.ds(32, 16)] = vcs
    pltpu.sync_copy(stage, out_hbm.at[w])


def _combine(p_ref, out_ref):
    x = p_ref[...]
    n = jnp.sum(x[:, 0:16], axis=0, keepdims=True)
    tp = jnp.sum(x[:, 16:32], axis=0, keepdims=True)
    cs = jnp.sum(x[:, 32:48], axis=0, keepdims=True)
    total = jnp.sum(n)
    nsafe = jnp.maximum(n, 1.0)
    term = jnp.where(n > 0.0, (n / total) * ((cs - tp) / nsafe) ** 2, 0.0)
    out_ref[...] = jnp.full((1, 1), jnp.sqrt(jnp.sum(term)), jnp.float32)


def kernel(probas, labels, matchings):
    labels32 = labels.astype(jnp.int32)
    match32 = matchings.astype(jnp.int32)

    mesh = plsc.VectorSubcoreMesh(core_axis_name="c", subcore_axis_name="s",
                                  num_cores=2, num_subcores=16)
    partials = pl.kernel(
        _sc_body,
        out_type=jax.ShapeDtypeStruct((_NW, 48), jnp.float32),
        mesh=mesh,
        compiler_params=pltpu.CompilerParams(needs_layout_passes=False),
        scratch_types=[
            pltpu.VMEM((_CR, _C), jnp.float32),
            pltpu.VMEM((_CR, _C), jnp.float32),
            pltpu.VMEM((_CR,), jnp.int32),
            pltpu.VMEM((_CR,), jnp.int32),
            pltpu.VMEM((_CR,), jnp.int32),
            pltpu.VMEM((_CR,), jnp.int32),
            pltpu.VMEM((256,), jnp.float32),
            pltpu.VMEM((256,), jnp.float32),
            pltpu.VMEM((256,), jnp.float32),
            pltpu.VMEM((48,), jnp.float32),
            pltpu.SemaphoreType.DMA,
            pltpu.SemaphoreType.DMA,
        ],
    )(probas, labels32, match32)

    out = pl.pallas_call(
        _combine,
        in_specs=[pl.BlockSpec((_NW, 48), lambda: (0, 0))],
        out_specs=pl.BlockSpec((1, 1), lambda: (0, 0)),
        out_shape=jax.ShapeDtypeStruct((1, 1), jnp.float32),
    )(partials)
    return out[0, 0]
